# baseline (speedup 1.0000x reference)
import jax
import jax.numpy as jnp
from jax.experimental import pallas as pl


B, N, M, C = 2, 4096, 4096, 3
BN = 1024
K = 8
MC = 1280


def _chamfer_body(a_ref, bt_ref, x1_ref, x2t_ref, d1_ref, d2_ref):
    nb = pl.program_id(1)
    t = jnp.dot(a_ref[0], bt_ref[0],
                preferred_element_type=jnp.float32,
                precision=jax.lax.Precision.HIGHEST)
    acc = None
    for c in range(C):
        av = x1_ref[0, :, c:c + 1]
        bv = x2t_ref[0, c:c + 1, :]
        diff = av - bv
        sq = diff * diff
        acc = sq if acc is None else acc + sq

    d1_ref[0, 0, :] = jnp.minimum(jnp.min(t, axis=1), jnp.min(acc, axis=1))
    part_l = jnp.min(t, axis=0)
    part_r = jnp.min(acc, axis=0)

    @pl.when(nb == 0)
    def _init():
        d2_ref[0, 0, :MC] = part_l
        d2_ref[0, 0, MC:] = part_r

    @pl.when(nb != 0)
    def _accum():
        d2_ref[0, 0, :MC] = jnp.minimum(d2_ref[0, 0, :MC], part_l)
        d2_ref[0, 0, MC:] = jnp.minimum(d2_ref[0, 0, MC:], part_r)


@jax.jit
def kernel(xyz1, xyz2):
    f32 = jnp.float32
    n1 = jnp.sum(xyz1 * xyz1, axis=-1, keepdims=True)
    n2 = jnp.sum(xyz2 * xyz2, axis=-1, keepdims=True)
    a_aug = jnp.concatenate(
        [-2.0 * xyz1, jnp.ones_like(n1), n1,
         jnp.zeros((B, N, 3), f32)], axis=-1)
    b_aug = jnp.concatenate(
        [xyz2, n2, jnp.ones_like(n2),
         jnp.zeros((B, M, 3), f32)], axis=-1)
    bt = jnp.transpose(b_aug[:, :MC, :], (0, 2, 1))
    x2t = jnp.transpose(xyz2[:, MC:, :], (0, 2, 1))

    grid = (B, N // BN)
    d1, d2 = pl.pallas_call(
        _chamfer_body,
        grid=grid,
        in_specs=[
            pl.BlockSpec((1, BN, K), lambda b, nb: (b, nb, 0)),
            pl.BlockSpec((1, K, MC), lambda b, nb: (b, 0, 0)),
            pl.BlockSpec((1, BN, C), lambda b, nb: (b, nb, 0)),
            pl.BlockSpec((1, C, M - MC), lambda b, nb: (b, 0, 0)),
        ],
        out_specs=[
            pl.BlockSpec((1, 1, BN), lambda b, nb: (b, 0, nb)),
            pl.BlockSpec((1, 1, M), lambda b, nb: (b, 0, 0)),
        ],
        out_shape=[
            jax.ShapeDtypeStruct((B, 1, N), f32),
            jax.ShapeDtypeStruct((B, 1, M), f32),
        ],
    )(a_aug, bt, xyz1, x2t)
    return d1.reshape(B, N), d2.reshape(B, M)
